# baseline (speedup 1.0000x reference)
import functools

import jax
import jax.numpy as jnp
import numpy as np
from jax import lax
from jax.experimental import pallas as pl
from jax.experimental.pallas import tpu as pltpu
from jax.experimental.pallas import tpu_sc as plsc

N_NODES = 10000
N_PAD = 10240
N_EDGES = 320000
D_FEAT = 128
N_WORKERS = 32
N_TILES = 16
CHUNK = 128
N_PHASES = 5
CHUNKS_PER_PHASE = 16
E_PAD = N_WORKERS * N_PHASES * CHUNKS_PER_PHASE * CHUNK
N_CHUNKS = N_EDGES // CHUNK
PAD_CHUNKS = (E_PAD - N_EDGES) // CHUNK
ROWS_PER_TILE = N_PAD // N_TILES
LANES = 16

_fill = np.arange(E_PAD - N_EDGES, dtype=np.int32)
_PAD_EDGES = np.stack([(_fill * 37) % N_NODES,
                       N_NODES + _fill % (N_PAD - N_NODES)]
                      ).reshape(2, PAD_CHUNKS, CHUNK)

_mesh = plsc.VectorSubcoreMesh(core_axis_name="c", subcore_axis_name="s")


@functools.partial(
    pl.kernel,
    out_type=(jax.ShapeDtypeStruct((N_NODES, D_FEAT), jnp.float32),
              jax.ShapeDtypeStruct((N_NODES, D_FEAT), jnp.float32)),
    mesh=_mesh,
    scratch_types=[
        pltpu.VMEM((2, 2, CHUNKS_PER_PHASE, CHUNK), jnp.int32),
        pltpu.VMEM((CHUNK, D_FEAT), jnp.float32),
        pltpu.VMEM((CHUNK, D_FEAT), jnp.float32),
        pltpu.VMEM_SHARED((N_PAD, D_FEAT), jnp.float32),
        pltpu.SemaphoreType.DMA,
        pltpu.SemaphoreType.DMA,
        pltpu.SemaphoreType.DMA,
        pltpu.SemaphoreType.DMA,
        pltpu.SemaphoreType.DMA,
        pltpu.SemaphoreType.DMA,
        pltpu.SemaphoreType.DMA,
    ],
)
def _feature_sum(emb, esd_hbm, out0, out1,
                 idx, rows_a, rows_b, acc, sem_a, sem_b, si0, si1, sz,
                 ssa, ssb):
    c = lax.axis_index("c")
    s = lax.axis_index("s")
    wid = c * N_TILES + s
    row0 = s * ROWS_PER_TILE
    isems = (si0, si1)

    def fire_stage(ph, buf):
        chunk0 = wid * (N_PHASES * CHUNKS_PER_PHASE) + ph * CHUNKS_PER_PHASE
        pltpu.async_copy(esd_hbm.at[0, pl.ds(chunk0, CHUNKS_PER_PHASE)],
                         idx.at[buf, 0], isems[buf])
        pltpu.async_copy(esd_hbm.at[1, pl.ds(chunk0, CHUNKS_PER_PHASE)],
                         idx.at[buf, 1], isems[buf])

    def wait_stage(buf):
        pltpu.make_async_copy(
            esd_hbm.at[0, pl.ds(0, CHUNKS_PER_PHASE)],
            idx.at[buf, 0], isems[buf]).wait()
        pltpu.make_async_copy(
            esd_hbm.at[1, pl.ds(0, CHUNKS_PER_PHASE)],
            idx.at[buf, 1], isems[buf]).wait()

    fire_stage(0, 0)
    fire_stage(1, 1)

    @pl.loop(0, CHUNK)
    def _zrow(i):
        for u in range(D_FEAT // LANES):
            rows_a[i, pl.ds(u * LANES, LANES)] = jnp.zeros((LANES,), jnp.float32)

    for r in range(ROWS_PER_TILE // CHUNK):
        pltpu.async_copy(rows_a, acc.at[pl.ds(row0 + r * CHUNK, CHUNK)], sz)

    for r in range(ROWS_PER_TILE // CHUNK):
        pltpu.make_async_copy(
            rows_a, acc.at[pl.ds(row0 + r * CHUNK, CHUNK)], sz).wait()

    plsc.subcore_barrier()

    for ph in range(N_PHASES):
        buf = ph % 2
        wait_stage(buf)
        pltpu.async_copy(emb.at[idx.at[buf, 0, 0]], rows_a, sem_a)
        pltpu.async_copy(emb.at[idx.at[buf, 0, 1]], rows_b, sem_b)

        @pl.loop(0, CHUNKS_PER_PHASE, step=2)
        def _chunk(j):
            pltpu.make_async_copy(
                emb.at[idx.at[buf, 0, j]], rows_a, sem_a).wait()
            pltpu.async_copy(rows_a, acc.at[idx.at[buf, 1, j]], ssa,
                             add=True)
            pltpu.make_async_copy(
                emb.at[idx.at[buf, 0, j + 1]], rows_b, sem_b).wait()
            pltpu.async_copy(rows_b, acc.at[idx.at[buf, 1, j + 1]], ssb,
                             add=True)
            pltpu.make_async_copy(
                rows_a, acc.at[idx.at[buf, 1, j]], ssa).wait()

            @pl.when(j + 2 < CHUNKS_PER_PHASE)
            def _():
                pltpu.async_copy(emb.at[idx.at[buf, 0, j + 2]], rows_a, sem_a)

            pltpu.make_async_copy(
                rows_b, acc.at[idx.at[buf, 1, j + 1]], ssb).wait()

            @pl.when(j + 3 < CHUNKS_PER_PHASE)
            def _():
                pltpu.async_copy(emb.at[idx.at[buf, 0, j + 3]], rows_b, sem_b)

        if ph + 2 < N_PHASES:
            fire_stage(ph + 2, buf)

    plsc.subcore_barrier()

    last_rows = N_NODES - (N_TILES - 1) * ROWS_PER_TILE

    @pl.when(c == 0)
    def _():
        @pl.when(s < N_TILES - 1)
        def _():
            pltpu.sync_copy(acc.at[pl.ds(row0, ROWS_PER_TILE)],
                            out0.at[pl.ds(row0, ROWS_PER_TILE)])

        @pl.when(s == N_TILES - 1)
        def _():
            pltpu.sync_copy(acc.at[pl.ds(row0, last_rows)],
                            out0.at[pl.ds(row0, last_rows)])

    @pl.when(c == 1)
    def _():
        @pl.when(s < N_TILES - 1)
        def _():
            pltpu.sync_copy(acc.at[pl.ds(row0, ROWS_PER_TILE)],
                            out1.at[pl.ds(row0, ROWS_PER_TILE)])

        @pl.when(s == N_TILES - 1)
        def _():
            pltpu.sync_copy(acc.at[pl.ds(row0, last_rows)],
                            out1.at[pl.ds(row0, last_rows)])


def _add_body(a_ref, b_ref, o_ref):
    o_ref[...] = a_ref[...] + b_ref[...]


_merge = pl.pallas_call(
    _add_body,
    out_shape=jax.ShapeDtypeStruct((N_NODES, D_FEAT), jnp.float32),
    grid=(5,),
    in_specs=[pl.BlockSpec((2000, D_FEAT), lambda i: (i, 0)),
              pl.BlockSpec((2000, D_FEAT), lambda i: (i, 0))],
    out_specs=pl.BlockSpec((2000, D_FEAT), lambda i: (i, 0)),
)


def kernel(graph_embedding, edge_index):
    ei3 = edge_index.astype(jnp.int32).reshape(2, N_CHUNKS, CHUNK)
    esd = jnp.concatenate([ei3, jnp.asarray(_PAD_EDGES)], axis=1)
    p0, p1 = _feature_sum(graph_embedding, esd)
    return _merge(p0, p1)


# speedup vs baseline: 1.2382x; 1.2382x over previous
import functools

import jax
import jax.numpy as jnp
import numpy as np
from jax import lax
from jax.experimental import pallas as pl
from jax.experimental.pallas import tpu as pltpu
from jax.experimental.pallas import tpu_sc as plsc

N_NODES = 10000
N_PAD = 10240
N_EDGES = 320000
D_FEAT = 128
N_WORKERS = 32
N_TILES = 16
CHUNK = 128
N_PHASES = 5
CHUNKS_PER_PHASE = 16
E_PAD = N_WORKERS * N_PHASES * CHUNKS_PER_PHASE * CHUNK
N_CHUNKS = N_EDGES // CHUNK
PAD_CHUNKS = (E_PAD - N_EDGES) // CHUNK
ROWS_PER_TILE = N_PAD // N_TILES
LANES = 16

_fill = np.arange(E_PAD - N_EDGES, dtype=np.int32)
_PAD_EDGES = np.stack([(_fill * 37) % N_NODES,
                       N_NODES + _fill % (N_PAD - N_NODES)]
                      ).reshape(2, PAD_CHUNKS, CHUNK)

_mesh = plsc.VectorSubcoreMesh(core_axis_name="c", subcore_axis_name="s")


@functools.partial(
    pl.kernel,
    out_type=(jax.ShapeDtypeStruct((N_NODES, D_FEAT), jnp.float32),
              jax.ShapeDtypeStruct((N_NODES, D_FEAT), jnp.float32)),
    mesh=_mesh,
    scratch_types=[
        pltpu.VMEM((2, 2, CHUNKS_PER_PHASE, CHUNK), jnp.int32),
        pltpu.VMEM((CHUNK, D_FEAT), jnp.float32),
        pltpu.VMEM((CHUNK, D_FEAT), jnp.float32),
        pltpu.VMEM_SHARED((N_PAD, D_FEAT), jnp.float32),
        pltpu.SemaphoreType.DMA,
        pltpu.SemaphoreType.DMA,
        pltpu.SemaphoreType.DMA,
        pltpu.SemaphoreType.DMA,
        pltpu.SemaphoreType.DMA,
        pltpu.SemaphoreType.DMA,
        pltpu.SemaphoreType.DMA,
    ],
)
def _feature_sum(emb, esd_hbm, out0, out1,
                 idx, rows_a, rows_b, acc, sem_a, sem_b, si0, si1, sz,
                 ssa, ssb):
    c = lax.axis_index("c")
    s = lax.axis_index("s")
    wid = c * N_TILES + s
    row0 = s * ROWS_PER_TILE
    isems = (si0, si1)

    def fire_stage(ph, buf):
        chunk0 = wid * (N_PHASES * CHUNKS_PER_PHASE) + ph * CHUNKS_PER_PHASE
        pltpu.async_copy(esd_hbm.at[0, pl.ds(chunk0, CHUNKS_PER_PHASE)],
                         idx.at[buf, 0], isems[buf])
        pltpu.async_copy(esd_hbm.at[1, pl.ds(chunk0, CHUNKS_PER_PHASE)],
                         idx.at[buf, 1], isems[buf])

    def wait_stage(buf):
        pltpu.make_async_copy(
            esd_hbm.at[0, pl.ds(0, CHUNKS_PER_PHASE)],
            idx.at[buf, 0], isems[buf]).wait()
        pltpu.make_async_copy(
            esd_hbm.at[1, pl.ds(0, CHUNKS_PER_PHASE)],
            idx.at[buf, 1], isems[buf]).wait()

    fire_stage(0, 0)
    fire_stage(1, 1)

    @pl.loop(0, CHUNK)
    def _zrow(i):
        for u in range(D_FEAT // LANES):
            rows_a[i, pl.ds(u * LANES, LANES)] = jnp.zeros((LANES,), jnp.float32)

    for r in range(ROWS_PER_TILE // CHUNK):
        pltpu.async_copy(rows_a, acc.at[pl.ds(row0 + r * CHUNK, CHUNK)], sz)

    for r in range(ROWS_PER_TILE // CHUNK):
        pltpu.make_async_copy(
            rows_a, acc.at[pl.ds(row0 + r * CHUNK, CHUNK)], sz).wait()

    plsc.subcore_barrier()

    for ph in range(N_PHASES):
        buf = ph % 2
        wait_stage(buf)
        pltpu.async_copy(emb.at[idx.at[buf, 0, 0]], rows_a, sem_a)

        @pl.loop(0, CHUNKS_PER_PHASE, step=2)
        def _chunk(j):
            pltpu.async_copy(emb.at[idx.at[buf, 0, j + 1]], rows_b, sem_b)
            pltpu.make_async_copy(
                emb.at[idx.at[buf, 0, j]], rows_a, sem_a).wait()
            pltpu.sync_copy(rows_a, acc.at[idx.at[buf, 1, j]], add=True)

            @pl.when(j + 2 < CHUNKS_PER_PHASE)
            def _():
                pltpu.async_copy(emb.at[idx.at[buf, 0, j + 2]], rows_a, sem_a)

            pltpu.make_async_copy(
                emb.at[idx.at[buf, 0, j + 1]], rows_b, sem_b).wait()
            pltpu.sync_copy(rows_b, acc.at[idx.at[buf, 1, j + 1]], add=True)

        if ph + 2 < N_PHASES:
            fire_stage(ph + 2, buf)

    plsc.subcore_barrier()

    last_rows = N_NODES - (N_TILES - 1) * ROWS_PER_TILE

    @pl.when(c == 0)
    def _():
        @pl.when(s < N_TILES - 1)
        def _():
            pltpu.sync_copy(acc.at[pl.ds(row0, ROWS_PER_TILE)],
                            out0.at[pl.ds(row0, ROWS_PER_TILE)])

        @pl.when(s == N_TILES - 1)
        def _():
            pltpu.sync_copy(acc.at[pl.ds(row0, last_rows)],
                            out0.at[pl.ds(row0, last_rows)])

    @pl.when(c == 1)
    def _():
        @pl.when(s < N_TILES - 1)
        def _():
            pltpu.sync_copy(acc.at[pl.ds(row0, ROWS_PER_TILE)],
                            out1.at[pl.ds(row0, ROWS_PER_TILE)])

        @pl.when(s == N_TILES - 1)
        def _():
            pltpu.sync_copy(acc.at[pl.ds(row0, last_rows)],
                            out1.at[pl.ds(row0, last_rows)])


def _add_body(a_ref, b_ref, o_ref):
    o_ref[...] = a_ref[...] + b_ref[...]


_merge = pl.pallas_call(
    _add_body,
    out_shape=jax.ShapeDtypeStruct((N_NODES, D_FEAT), jnp.float32),
    grid=(5,),
    in_specs=[pl.BlockSpec((2000, D_FEAT), lambda i: (i, 0)),
              pl.BlockSpec((2000, D_FEAT), lambda i: (i, 0))],
    out_specs=pl.BlockSpec((2000, D_FEAT), lambda i: (i, 0)),
)


def kernel(graph_embedding, edge_index):
    ei3 = edge_index.astype(jnp.int32).reshape(2, N_CHUNKS, CHUNK)
    esd = jnp.concatenate([ei3, jnp.asarray(_PAD_EDGES)], axis=1)
    p0, p1 = _feature_sum(graph_embedding, esd)
    return _merge(p0, p1)
